# baseline (speedup 1.0000x reference)
import functools

import jax
import jax.numpy as jnp
from jax import lax
from jax.experimental import pallas as pl
from jax.experimental.pallas import tpu as pltpu
from jax.experimental.pallas import tpu_sc as plsc

N = 10000
NP = 10240
E = 160000
NB = 90
EB = 112
E_PAD = 16 * NB * EB
BM = 2048
ROWS = NP // 16


JOBS = 12


def _mm_kernel(a_ref, w_ref, buf_ref, o_ref):
    k = pl.program_id(2)

    @pl.when(k == 0)
    def _():
        o_ref[...] = jnp.zeros_like(o_ref)

    o_ref[...] += jnp.dot(a_ref[...], w_ref[...],
                          preferred_element_type=jnp.float32)


def _mm_elu_kernel(a_ref, w_ref, o_ref):
    k = pl.program_id(2)

    @pl.when(k == 0)
    def _():
        o_ref[...] = jnp.zeros_like(o_ref)

    a = a_ref[...]
    a = jnp.where(a > 0.0, a, jnp.exp(a) - 1.0)
    o_ref[...] += jnp.dot(a, w_ref[...], preferred_element_type=jnp.float32)


def _mm_x(x, w, c_out):
    k_dim = x.shape[1]

    def kern(a_ref, w_ref, o_ref):
        _mm_kernel(a_ref, w_ref, None, o_ref)

    return pl.pallas_call(
        kern,
        grid=(NP // BM, c_out, 1),
        in_specs=[
            pl.BlockSpec((BM, k_dim), lambda i, j, k: (i, 0)),
            pl.BlockSpec((k_dim, 128), lambda i, j, k: (0, j)),
        ],
        out_specs=pl.BlockSpec((BM, 128),
                               lambda i, j, k: (j * (NP // BM) + i, 0)),
        out_shape=jax.ShapeDtypeStruct((JOBS * NP, 128), jnp.float32),
    )(x, w)


def _mm_elu_kernel4(a_ref, w_ref, o_ref):
    k = pl.program_id(3)

    @pl.when(k == 0)
    def _():
        o_ref[...] = jnp.zeros_like(o_ref)

    a = a_ref[...]
    a = jnp.where(a > 0.0, a, jnp.exp(a) - 1.0)
    o_ref[...] += jnp.dot(a, w_ref[...], preferred_element_type=jnp.float32)


def _mm_flat(h, w, c_in, c_out):
    gm = NP // BM
    return pl.pallas_call(
        _mm_elu_kernel4,
        grid=(3, gm, c_out, c_in),
        in_specs=[
            pl.BlockSpec((BM, 128),
                         lambda g, i, j, k, c=c_in: ((g * c + k) * (NP // BM) + i, 0)),
            pl.BlockSpec((128, 128), lambda g, i, j, k: (k, j)),
        ],
        out_specs=pl.BlockSpec(
            (BM, 128), lambda g, i, j, k, c=c_out: ((g * c + j) * (NP // BM) + i, 0)),
        out_shape=jax.ShapeDtypeStruct((JOBS * NP, 128), jnp.float32),
    )(h, w)


def _elu_kernel(a_ref, o_ref):
    a = a_ref[...]
    o_ref[...] = jnp.where(a > 0.0, a, jnp.exp(a) - 1.0)


def _elu1(h):
    return pl.pallas_call(
        _elu_kernel,
        grid=(NP // BM,),
        in_specs=[pl.BlockSpec((BM, 128), lambda i: (i, 0))],
        out_specs=pl.BlockSpec((BM, 128), lambda i: (i, 0)),
        out_shape=jax.ShapeDtypeStruct((NP, 128), jnp.float32),
    )(h)


def _mm_flat1(h, w, c_in, c_out):
    return pl.pallas_call(
        _mm_elu_kernel,
        grid=(NP // BM, c_out, c_in),
        in_specs=[
            pl.BlockSpec((BM, 128), lambda i, j, k: (k * (NP // BM) + i, 0)),
            pl.BlockSpec((128, 128), lambda i, j, k: (k, j)),
        ],
        out_specs=pl.BlockSpec(
            (BM, 128), lambda i, j, k: (j * (NP // BM) + i, 0)),
        out_shape=jax.ShapeDtypeStruct((JOBS * NP, 128), jnp.float32),
    )(h, w)


def _make_spmm():
    mesh = plsc.VectorSubcoreMesh(core_axis_name="c", subcore_axis_name="s")

    @functools.partial(
        pl.kernel,
        mesh=mesh,
        out_type=jax.ShapeDtypeStruct((JOBS * NP, 128), jnp.float32),
        scratch_types=[
            [pltpu.VMEM((2 * EB,), jnp.float32)] * 3,
            [pltpu.VMEM((EB,), jnp.int32)] * 3,
            [pltpu.VMEM((EB,), jnp.int32)] * 3,
            [pltpu.VMEM((EB, 128), jnp.float32)] * 3,
            pltpu.VMEM((32,), jnp.int32),
            pltpu.VMEM_SHARED((NP, 128), jnp.float32),
            [pltpu.SemaphoreType.DMA] * 3,
            [pltpu.SemaphoreType.DMA] * 3,
            [pltpu.SemaphoreType.DMA] * 3,
            [pltpu.SemaphoreType.DMA] * 3,
        ],
    )
    def spmm(sv4, dst4, sup, zeros_hbm, cc_hbm, out,
             meta, idxb, dstv, rows, cc_v, accum, msem, dsem, gsem, scsem):
        cid = lax.axis_index("c")
        sid = lax.axis_index("s")
        row0 = sid * ROWS

        pltpu.sync_copy(cc_hbm, cc_v)
        ev = cc_v[pl.ds(0, 16)]
        n_jobs = ev[0]
        jdiv = ev[1]
        gstr = ev[2]
        esl_b = ev[3]
        gc_b = ev[4]
        out_b = ev[5]
        rounds = (n_jobs + 1) // 2

        def adjust(ms, islot, off):
            for q in range(EB // 16):
                sl = pl.ds(q * 16, 16)
                idxb[islot][sl] = meta[ms][sl].astype(jnp.int32) + off

        def scale(rs, ms):
            def scale16(e16, c2):
                fv = meta[ms][pl.ds(EB + e16 * 16, 16)]
                for i in range(16):
                    sc = fv[i]
                    e = e16 * 16 + i
                    for q in range(128 // 16):
                        sl = pl.ds(q * 16, 16)
                        rows[rs][e, sl] = rows[rs][e, sl] * sc
                return c2

            lax.fori_loop(0, EB // 16, scale16, 0)

        def round_body(r, carry):
            j = r * 2 + cid
            active = j < n_jobs
            esl = esl_b + j // jdiv
            off = (gc_b + (j // jdiv) * gstr + j % jdiv) * NP

            pltpu.sync_copy(zeros_hbm.at[pl.ds(row0, ROWS)],
                            accum.at[pl.ds(row0, ROWS)])
            plsc.subcore_barrier()

            @pl.when(active)
            def _():
                pltpu.async_copy(sv4.at[esl, sid, 0], meta[0], msem[0])
                pltpu.async_copy(sv4.at[esl, sid, 1], meta[1], msem[1])
                pltpu.async_copy(dst4.at[esl, sid, 0], dstv[0], dsem[0])
                pltpu.make_async_copy(sv4.at[esl, sid, 0], meta[0], msem[0]).wait()
                adjust(0, 0, off)
                pltpu.async_copy(sup.at[idxb[0]], rows[0], gsem[0])

                def block(bq, s, first2, nog1, nog2):
                    b = bq * 3 + s
                    s1 = (s + 1) % 3
                    s2 = (s + 2) % 3
                    if not nog1:
                        pltpu.make_async_copy(sv4.at[esl, sid, b + 1],
                                              meta[s1], msem[s1]).wait()
                        adjust(s1, s1, off)
                        if not first2:
                            pltpu.make_async_copy(
                                rows[s1], accum.at[dstv[s1]],
                                scsem[s1]).wait()
                        pltpu.async_copy(sup.at[idxb[s1]], rows[s1],
                                         gsem[s1])
                        pltpu.async_copy(dst4.at[esl, sid, b + 1], dstv[s1],
                                         dsem[s1])
                    if not nog2:
                        pltpu.async_copy(sv4.at[esl, sid, b + 2], meta[s2],
                                         msem[s2])
                    pltpu.make_async_copy(sup.at[idxb[s]], rows[s],
                                          gsem[s]).wait()
                    scale(s, s)
                    pltpu.make_async_copy(dst4.at[esl, sid, b], dstv[s],
                                          dsem[s]).wait()
                    pltpu.async_copy(rows[s], accum.at[dstv[s]],
                                     scsem[s], add=True)

                for s in range(3):
                    block(0, s, first2=(s < 2), nog1=False, nog2=False)

                def triple(bq, c2):
                    for s in range(3):
                        block(bq, s, first2=False, nog1=False, nog2=False)
                    return c2

                lax.fori_loop(1, NB // 3 - 1, triple, 0)

                for s in range(3):
                    b = NB - 3 + s
                    block(NB // 3 - 1, s, first2=False,
                          nog1=(b + 1 >= NB), nog2=(b + 2 >= NB))

                for s in range(3):
                    pltpu.make_async_copy(rows[s], accum.at[dstv[s]],
                                          scsem[s]).wait()

            plsc.subcore_barrier()

            @pl.when(active)
            def _():
                pltpu.sync_copy(accum.at[pl.ds(row0, ROWS)],
                                out.at[pl.ds((out_b + j) * NP + row0, ROWS)])

            return carry

        lax.fori_loop(0, rounds, round_body, 0)

    return spmm


_spmm = _make_spmm()


def _prep_edges(idx, val):
    pad = E_PAD - E
    spread = (jnp.arange(pad, dtype=jnp.int32) * 7) % N
    src = jnp.concatenate([idx[1].astype(jnp.int32), spread])
    dst = jnp.concatenate([idx[0].astype(jnp.int32), spread])
    v = jnp.concatenate([val, jnp.zeros((pad,), jnp.float32)])
    sv = jnp.stack([src.astype(jnp.float32).reshape(16, NB, EB),
                    v.reshape(16, NB, EB)],
                   axis=2).reshape(16, NB, 2 * EB)
    return sv, dst.reshape(16, NB, EB)


def _params(n_jobs, jdiv, gstr, esl_b=0, gc_b=0, out_b=0):
    return jnp.asarray([n_jobs, jdiv, gstr, esl_b, gc_b, out_b] + [0] * 26,
                       jnp.int32)


def kernel(x, adj_idx, adj_val, adj_knn_idx, adj_knn_val,
           adj_diff_idx, adj_diff_val, W1, W2, W3):
    zeros = jnp.zeros((NP, 128), jnp.float32)
    x = jnp.pad(x, ((0, NP - N), (0, 0)))
    prepped = [_prep_edges(i, v) for i, v in
               ((adj_idx, adj_val), (adj_knn_idx, adj_knn_val),
                (adj_diff_idx, adj_diff_val))]
    sv4 = jnp.stack([p[0] for p in prepped])
    dst4 = jnp.stack([p[1] for p in prepped])

    cc1 = _params(12, 4, 0)
    sup1 = _mm_x(x, W1, 4)
    h1 = _spmm(sv4, dst4, sup1, zeros, cc1)

    sup2 = _mm_flat(h1, W2, 4, 2)
    outs = []
    for g in range(3):
        h2_g = _spmm(sv4, dst4, sup2, zeros,
                     _params(2, 2, 0, esl_b=g, gc_b=2 * g, out_b=0))
        sup3_g = _mm_flat1(h2_g, W3, 2, 1)
        h3_g = _spmm(sv4, dst4, sup3_g, zeros,
                     _params(1, 1, 0, esl_b=g, gc_b=0, out_b=0))
        outs.append(_elu1(h3_g)[:N])
    return tuple(outs)


# speedup vs baseline: 1.0878x; 1.0878x over previous
import functools

import jax
import jax.numpy as jnp
from jax import lax
from jax.experimental import pallas as pl
from jax.experimental.pallas import tpu as pltpu
from jax.experimental.pallas import tpu_sc as plsc

N = 10000
NP = 10240
E = 160000
NB = 90
EB = 112
E_PAD = 16 * NB * EB
BM = 2048
ROWS = NP // 16


JOBS = 12


def _mm_kernel(a_ref, w_ref, buf_ref, o_ref):
    k = pl.program_id(2)

    @pl.when(k == 0)
    def _():
        o_ref[...] = jnp.zeros_like(o_ref)

    o_ref[...] += jnp.dot(a_ref[...], w_ref[...],
                          preferred_element_type=jnp.float32)


def _mm_elu_kernel(a_ref, w_ref, buf_ref, o_ref):
    k = pl.program_id(2)

    @pl.when(k == 0)
    def _():
        o_ref[...] = jnp.zeros_like(o_ref)

    a = a_ref[...]
    a = jnp.where(a > 0.0, a, jnp.exp(a) - 1.0)
    o_ref[...] += jnp.dot(a, w_ref[...], preferred_element_type=jnp.float32)


def _mm_x(x, w, c_out):
    k_dim = x.shape[1]

    def kern(a_ref, w_ref, o_ref):
        _mm_kernel(a_ref, w_ref, None, o_ref)

    return pl.pallas_call(
        kern,
        grid=(NP // BM, c_out, 1),
        in_specs=[
            pl.BlockSpec((BM, k_dim), lambda i, j, k: (i, 0)),
            pl.BlockSpec((k_dim, 128), lambda i, j, k: (0, j)),
        ],
        out_specs=pl.BlockSpec((BM, 128),
                               lambda i, j, k: (j * (NP // BM) + i, 0)),
        out_shape=jax.ShapeDtypeStruct((JOBS * NP, 128), jnp.float32),
    )(x, w)


def _mm_elu_kernel4(a_ref, w_ref, o_ref):
    k = pl.program_id(3)

    @pl.when(k == 0)
    def _():
        o_ref[...] = jnp.zeros_like(o_ref)

    a = a_ref[...]
    a = jnp.where(a > 0.0, a, jnp.exp(a) - 1.0)
    o_ref[...] += jnp.dot(a, w_ref[...], preferred_element_type=jnp.float32)


def _mm_flat(h, w, c_in, c_out):
    gm = NP // BM
    return pl.pallas_call(
        _mm_elu_kernel4,
        grid=(3, gm, c_out, c_in),
        in_specs=[
            pl.BlockSpec((BM, 128),
                         lambda g, i, j, k, c=c_in: ((g * c + k) * (NP // BM) + i, 0)),
            pl.BlockSpec((128, 128), lambda g, i, j, k: (k, j)),
        ],
        out_specs=pl.BlockSpec(
            (BM, 128), lambda g, i, j, k, c=c_out: ((g * c + j) * (NP // BM) + i, 0)),
        out_shape=jax.ShapeDtypeStruct((JOBS * NP, 128), jnp.float32),
    )(h, w)


def _elu_kernel(a_ref, o_ref):
    a = a_ref[...]
    o_ref[...] = jnp.where(a > 0.0, a, jnp.exp(a) - 1.0)


def _elu3(h):
    return pl.pallas_call(
        _elu_kernel,
        grid=(3 * NP // BM,),
        in_specs=[pl.BlockSpec((BM, 128), lambda i: (i, 0))],
        out_specs=pl.BlockSpec((BM, 128), lambda i: (i, 0)),
        out_shape=jax.ShapeDtypeStruct((3 * NP, 128), jnp.float32),
    )(h)


def _make_spmm():
    mesh = plsc.VectorSubcoreMesh(core_axis_name="c", subcore_axis_name="s")

    @functools.partial(
        pl.kernel,
        mesh=mesh,
        out_type=jax.ShapeDtypeStruct((JOBS * NP, 128), jnp.float32),
        scratch_types=[
            [pltpu.VMEM((2 * EB,), jnp.float32)] * 3,
            [pltpu.VMEM((EB,), jnp.int32)] * 3,
            [pltpu.VMEM((EB,), jnp.int32)] * 3,
            [pltpu.VMEM((EB, 128), jnp.float32)] * 3,
            pltpu.VMEM((32,), jnp.int32),
            pltpu.VMEM_SHARED((NP, 128), jnp.float32),
            [pltpu.SemaphoreType.DMA] * 3,
            [pltpu.SemaphoreType.DMA] * 3,
            [pltpu.SemaphoreType.DMA] * 3,
            [pltpu.SemaphoreType.DMA] * 3,
        ],
    )
    def spmm(sv4, dst4, sup, zeros_hbm, cc_hbm, out,
             meta, idxb, dstv, rows, cc_v, accum, msem, dsem, gsem, scsem):
        cid = lax.axis_index("c")
        sid = lax.axis_index("s")
        row0 = sid * ROWS

        pltpu.sync_copy(cc_hbm, cc_v)
        ev = cc_v[pl.ds(0, 16)]
        n_jobs = ev[0]
        jdiv = ev[1]
        gstr = ev[2]
        rounds = (n_jobs + 1) // 2

        def adjust(ms, islot, off):
            for q in range(EB // 16):
                sl = pl.ds(q * 16, 16)
                idxb[islot][sl] = meta[ms][sl].astype(jnp.int32) + off

        def scale(rs, ms):
            def scale16(e16, c2):
                fv = meta[ms][pl.ds(EB + e16 * 16, 16)]
                for i in range(16):
                    sc = fv[i]
                    e = e16 * 16 + i
                    for q in range(128 // 16):
                        sl = pl.ds(q * 16, 16)
                        rows[rs][e, sl] = rows[rs][e, sl] * sc
                return c2

            lax.fori_loop(0, EB // 16, scale16, 0)

        def round_body(r, carry):
            j = r * 2 + cid
            active = j < n_jobs
            esl = j // jdiv
            off = (esl * gstr + j % jdiv) * NP

            pltpu.sync_copy(zeros_hbm.at[pl.ds(row0, ROWS)],
                            accum.at[pl.ds(row0, ROWS)])
            plsc.subcore_barrier()

            @pl.when(active)
            def _():
                pltpu.async_copy(sv4.at[esl, sid, 0], meta[0], msem[0])
                pltpu.async_copy(sv4.at[esl, sid, 1], meta[1], msem[1])
                pltpu.async_copy(dst4.at[esl, sid, 0], dstv[0], dsem[0])
                pltpu.make_async_copy(sv4.at[esl, sid, 0], meta[0], msem[0]).wait()
                adjust(0, 0, off)
                pltpu.async_copy(sup.at[idxb[0]], rows[0], gsem[0])

                def block(bq, s, first2, nog1, nog2):
                    b = bq * 3 + s
                    s1 = (s + 1) % 3
                    s2 = (s + 2) % 3
                    if not nog1:
                        pltpu.make_async_copy(sv4.at[esl, sid, b + 1],
                                              meta[s1], msem[s1]).wait()
                        adjust(s1, s1, off)
                        if not first2:
                            pltpu.make_async_copy(
                                rows[s1], accum.at[dstv[s1]],
                                scsem[s1]).wait()
                        pltpu.async_copy(sup.at[idxb[s1]], rows[s1],
                                         gsem[s1])
                        pltpu.async_copy(dst4.at[esl, sid, b + 1], dstv[s1],
                                         dsem[s1])
                    if not nog2:
                        pltpu.async_copy(sv4.at[esl, sid, b + 2], meta[s2],
                                         msem[s2])
                    pltpu.make_async_copy(sup.at[idxb[s]], rows[s],
                                          gsem[s]).wait()
                    scale(s, s)
                    pltpu.make_async_copy(dst4.at[esl, sid, b], dstv[s],
                                          dsem[s]).wait()
                    pltpu.async_copy(rows[s], accum.at[dstv[s]],
                                     scsem[s], add=True)

                for s in range(3):
                    block(0, s, first2=(s < 2), nog1=False, nog2=False)

                def triple(bq, c2):
                    for s in range(3):
                        block(bq, s, first2=False, nog1=False, nog2=False)
                    return c2

                lax.fori_loop(1, NB // 3 - 1, triple, 0)

                for s in range(3):
                    b = NB - 3 + s
                    block(NB // 3 - 1, s, first2=False,
                          nog1=(b + 1 >= NB), nog2=(b + 2 >= NB))

                for s in range(3):
                    pltpu.make_async_copy(rows[s], accum.at[dstv[s]],
                                          scsem[s]).wait()

            plsc.subcore_barrier()

            @pl.when(active)
            def _():
                pltpu.sync_copy(accum.at[pl.ds(row0, ROWS)],
                                out.at[pl.ds(j * NP + row0, ROWS)])

            return carry

        lax.fori_loop(0, rounds, round_body, 0)

    return spmm


_spmm = _make_spmm()


def _prep_edges(idx, val):
    pad = E_PAD - E
    spread = (jnp.arange(pad, dtype=jnp.int32) * 7) % N
    src = jnp.concatenate([idx[1].astype(jnp.int32), spread])
    dst = jnp.concatenate([idx[0].astype(jnp.int32), spread])
    v = jnp.concatenate([val, jnp.zeros((pad,), jnp.float32)])
    sv = jnp.stack([src.astype(jnp.float32).reshape(16, NB, EB),
                    v.reshape(16, NB, EB)],
                   axis=2).reshape(16, NB, 2 * EB)
    return sv, dst.reshape(16, NB, EB)


def _params(n_jobs, jdiv, gstr):
    return jnp.asarray([n_jobs, jdiv, gstr] + [0] * 29, jnp.int32)


def kernel(x, adj_idx, adj_val, adj_knn_idx, adj_knn_val,
           adj_diff_idx, adj_diff_val, W1, W2, W3):
    zeros = jnp.zeros((NP, 128), jnp.float32)
    x = jnp.pad(x, ((0, NP - N), (0, 0)))
    prepped = [_prep_edges(i, v) for i, v in
               ((adj_idx, adj_val), (adj_knn_idx, adj_knn_val),
                (adj_diff_idx, adj_diff_val))]
    sv4 = jnp.stack([p[0] for p in prepped])
    dst4 = jnp.stack([p[1] for p in prepped])

    cc1 = _params(12, 4, 0)
    sup1 = _mm_x(x, W1, 4)
    h1 = _spmm(sv4, dst4, sup1, zeros, cc1)

    sup2 = _mm_flat(h1, W2, 4, 2)
    cc2 = _params(6, 2, 2)
    h2 = _spmm(sv4, dst4, sup2, zeros, cc2)

    sup3 = _mm_flat(h2, W3, 2, 1)
    cc3 = _params(3, 1, 1)
    h3 = _spmm(sv4, dst4, sup3, zeros, cc3)

    o = _elu3(h3)
    return (o[:N], o[NP:NP + N], o[2 * NP:2 * NP + N])
